# baseline (speedup 1.0000x reference)
import functools

import jax
import jax.numpy as jnp
from jax import lax
from jax.experimental import pallas as pl
from jax.experimental.pallas import tpu as pltpu
from jax.experimental.pallas import tpu_sc as plsc


def kernel(image_ids, embeddings_weight):
    (B,) = image_ids.shape
    V, D = embeddings_weight.shape
    info = plsc.get_sparse_core_info()
    NC, NS = info.num_cores, info.num_subcores
    NW = NC * NS
    assert B % NW == 0
    b_per_w = B // NW

    mesh = plsc.VectorSubcoreMesh(core_axis_name="c", subcore_axis_name="s")

    @functools.partial(
        pl.kernel,
        mesh=mesh,
        out_type=jax.ShapeDtypeStruct((B, D), jnp.float32),
        scratch_types=[
            pltpu.VMEM((b_per_w,), jnp.int32),
            pltpu.VMEM((b_per_w, D), jnp.float32),
            pltpu.SemaphoreType.DMA,
        ],
        compiler_params=pltpu.CompilerParams(use_tc_tiling_on_sc=False),
    )
    def gather_kernel(idx_hbm, table_hbm, out_hbm, idx_v, rows_v, sem):
        wid = lax.axis_index("s") * NC + lax.axis_index("c")
        base = wid * b_per_w
        pltpu.sync_copy(idx_hbm.at[pl.ds(base, b_per_w)], idx_v)
        pltpu.async_copy(table_hbm.at[idx_v], rows_v, sem).wait()
        pltpu.sync_copy(rows_v, out_hbm.at[pl.ds(base, b_per_w)])

    return gather_kernel(image_ids.astype(jnp.int32), embeddings_weight)


# speedup vs baseline: 4.7355x; 4.7355x over previous
import functools

import jax
import jax.numpy as jnp
from jax import lax
from jax.experimental import pallas as pl
from jax.experimental.pallas import tpu as pltpu
from jax.experimental.pallas import tpu_sc as plsc

_CHUNK = 16


def kernel(image_ids, embeddings_weight):
    (B,) = image_ids.shape
    V, D = embeddings_weight.shape
    info = plsc.get_sparse_core_info()
    NC, NS = info.num_cores, info.num_subcores
    NW = NC * NS
    assert B % (NW * _CHUNK) == 0
    b_per_w = B // NW

    mesh = plsc.VectorSubcoreMesh(core_axis_name="c", subcore_axis_name="s")

    @functools.partial(
        pl.kernel,
        mesh=mesh,
        out_type=jax.ShapeDtypeStruct((B, D), jnp.float32),
        scratch_types=[
            pltpu.VMEM((b_per_w,), jnp.int32),
            pltpu.VMEM((_CHUNK, D), jnp.float32),
            pltpu.SemaphoreType.DMA,
        ],
    )
    def gather_kernel(idx_hbm, table_hbm, out_hbm, idx_v, rows_v, sem):
        wid = lax.axis_index("s") * NC + lax.axis_index("c")
        base = wid * b_per_w
        pltpu.sync_copy(idx_hbm.at[pl.ds(base, b_per_w)], idx_v)

        @pl.loop(0, b_per_w // _CHUNK)
        def _chunk(c):
            off = c * _CHUNK
            idx_vec = idx_v[pl.ds(off, _CHUNK)]
            copies = []
            for j in range(_CHUNK):
                r = idx_vec[j]
                copies.append(
                    pltpu.async_copy(
                        table_hbm.at[pl.ds(r, 1)], rows_v.at[pl.ds(j, 1)], sem
                    )
                )
            for cp in copies:
                cp.wait()
            pltpu.sync_copy(rows_v, out_hbm.at[pl.ds(base + off, _CHUNK)])

    return gather_kernel(image_ids.astype(jnp.int32), embeddings_weight)


# speedup vs baseline: 4.9482x; 1.0449x over previous
import functools

import jax
import jax.numpy as jnp
from jax import lax
from jax.experimental import pallas as pl
from jax.experimental.pallas import tpu as pltpu
from jax.experimental.pallas import tpu_sc as plsc

_CHUNK = 16
_NBUF = 4


def kernel(image_ids, embeddings_weight):
    (B,) = image_ids.shape
    V, D = embeddings_weight.shape
    info = plsc.get_sparse_core_info()
    NC, NS = info.num_cores, info.num_subcores
    NW = NC * NS
    assert B % (NW * _CHUNK * _NBUF) == 0
    b_per_w = B // NW
    n_chunks = b_per_w // _CHUNK

    mesh = plsc.VectorSubcoreMesh(core_axis_name="c", subcore_axis_name="s")

    @functools.partial(
        pl.kernel,
        mesh=mesh,
        out_type=jax.ShapeDtypeStruct((B, D), jnp.float32),
        scratch_types=[
            pltpu.VMEM((b_per_w,), jnp.int32),
            pltpu.VMEM((_NBUF, _CHUNK, D), jnp.float32),
            pltpu.SemaphoreType.DMA,
            pltpu.SemaphoreType.DMA,
        ],
    )
    def gather_kernel(idx_hbm, table_hbm, out_hbm, idx_v, rows_v, sem_g, sem_w):
        wid = lax.axis_index("s") * NC + lax.axis_index("c")
        base = wid * b_per_w
        pltpu.sync_copy(idx_hbm.at[pl.ds(base, b_per_w)], idx_v)

        def fire_gather(k, b):
            for v in range(_CHUNK // 16):
                idx_vec = idx_v[pl.ds(k * _CHUNK + v * 16, 16)]
                for j in range(16):
                    r = idx_vec[j]
                    pltpu.async_copy(
                        table_hbm.at[pl.ds(r, 1)],
                        rows_v.at[b].at[pl.ds(v * 16 + j, 1)],
                        sem_g,
                    )

        def drain_gather():
            for j in range(_CHUNK):
                pltpu.make_async_copy(
                    table_hbm.at[pl.ds(0, 1)], rows_v.at[0].at[pl.ds(j, 1)], sem_g
                ).wait()

        def drain_write():
            pltpu.make_async_copy(
                rows_v.at[0], out_hbm.at[pl.ds(base, _CHUNK)], sem_w
            ).wait()

        for k0 in range(_NBUF - 1):
            fire_gather(k0, k0)

        @pl.loop(0, n_chunks // _NBUF)
        def _group(c):
            for b in range(_NBUF):
                k = c * _NBUF + b

                @pl.when(k >= 1)
                def _():
                    drain_write()

                @pl.when(k + (_NBUF - 1) < n_chunks)
                def _():
                    fire_gather(k + (_NBUF - 1), (b + _NBUF - 1) % _NBUF)

                drain_gather()
                pltpu.async_copy(
                    rows_v.at[b], out_hbm.at[pl.ds(base + k * _CHUNK, _CHUNK)], sem_w
                )

        drain_write()

    return gather_kernel(image_ids.astype(jnp.int32), embeddings_weight)


# speedup vs baseline: 5.0036x; 1.0112x over previous
import functools

import jax
import jax.numpy as jnp
from jax import lax
from jax.experimental import pallas as pl
from jax.experimental.pallas import tpu as pltpu
from jax.experimental.pallas import tpu_sc as plsc

_CHUNK = 16


def kernel(image_ids, embeddings_weight):
    (B,) = image_ids.shape
    V, D = embeddings_weight.shape
    info = plsc.get_sparse_core_info()
    NC, NS = info.num_cores, info.num_subcores
    NW = NC * NS
    assert B % (NW * _CHUNK) == 0
    b_per_w = B // NW
    n_chunks = b_per_w // _CHUNK

    mesh = plsc.VectorSubcoreMesh(core_axis_name="c", subcore_axis_name="s")

    @functools.partial(
        pl.kernel,
        mesh=mesh,
        out_type=jax.ShapeDtypeStruct((B, D), jnp.float32),
        scratch_types=[
            pltpu.VMEM((b_per_w,), jnp.int32),
            pltpu.VMEM((b_per_w, D), jnp.float32),
            pltpu.SemaphoreType.DMA,
        ],
    )
    def gather_kernel(idx_hbm, table_hbm, out_hbm, idx_v, rows_v, sem):
        wid = lax.axis_index("s") * NC + lax.axis_index("c")
        base = wid * b_per_w
        with jax.named_scope("stage_idx"):
            pltpu.sync_copy(idx_hbm.at[pl.ds(base, b_per_w)], idx_v)

        with jax.named_scope("fire_gathers"):

            @plsc.parallel_loop(0, n_chunks)
            def _fire(c):
                off = c * _CHUNK
                idx_vec = idx_v[pl.ds(off, _CHUNK)]
                for j in range(_CHUNK):
                    pltpu.async_copy(
                        table_hbm.at[pl.ds(idx_vec[j], 1)],
                        rows_v.at[pl.ds(off, _CHUNK)].at[pl.ds(j, 1)],
                        sem,
                    )

        with jax.named_scope("drain_gathers"):

            @pl.loop(0, n_chunks)
            def _drain(c):
                for j in range(_CHUNK):
                    pltpu.make_async_copy(
                        table_hbm.at[pl.ds(0, 1)],
                        rows_v.at[pl.ds(0, _CHUNK)].at[pl.ds(j, 1)],
                        sem,
                    ).wait()

        with jax.named_scope("writeback"):
            pltpu.sync_copy(rows_v, out_hbm.at[pl.ds(base, b_per_w)])

    return gather_kernel(image_ids.astype(jnp.int32), embeddings_weight)
